# Initial kernel scaffold; baseline (speedup 1.0000x reference)
import jax
import jax.numpy as jnp
from jax.experimental import pallas as pl


def kernel(boxes, labels, scores):
    raise NotImplementedError("write your pallas kernel here")


# speedup vs baseline: 12.0643x; 12.0643x over previous
import functools

import jax
import jax.numpy as jnp
from jax import lax
from jax.experimental import pallas as pl
from jax.experimental.pallas import tpu as pltpu

N = 5000
NUM_CLASSES = 8
IOU_THRESHOLD = 0.5
B = 128
NB = 40
NPAD = NB * B


def _pair_suppress(col, row):
    y1c, x1c, y2c, x2c, ac, lc = col
    y1r, x1r, y2r, x2r, ar, lr = row
    iy1 = jnp.maximum(y1c, y1r)
    ix1 = jnp.maximum(x1c, x1r)
    iy2 = jnp.minimum(y2c, y2r)
    ix2 = jnp.minimum(x2c, x2r)
    ih = jnp.clip(iy2 - iy1, 0.0)
    iw = jnp.clip(ix2 - ix1, 0.0)
    inter = ih * iw
    union = ac + ar - inter
    iou = inter / jnp.maximum(union, 1e-9)
    sup = (iou > IOU_THRESHOLD) & (lc == lr)
    return sup.astype(jnp.float32)


def _coords_row(rowd_ref, i):
    sl = pl.ds(pl.multiple_of(i * B, B), B)
    blk = rowd_ref[:, sl]
    yc = blk[0:1, :]
    xc = blk[1:2, :]
    h = jnp.abs(blk[2:3, :])
    w = jnp.abs(blk[3:4, :])
    lab = blk[4:5, :]
    return (yc - 0.5 * h, xc - 0.5 * w, yc + 0.5 * h, xc + 0.5 * w, h * w, lab)


def _coords_col(cold_ref, j):
    sl = pl.ds(pl.multiple_of(j * B, B), B)
    blk = cold_ref[sl, :]
    yc = blk[:, 0:1]
    xc = blk[:, 1:2]
    h = jnp.abs(blk[:, 2:3])
    w = jnp.abs(blk[:, 3:4])
    lab = blk[:, 4:5]
    return (yc - 0.5 * h, xc - 0.5 * w, yc + 0.5 * h, xc + 0.5 * w, h * w, lab)


def _nms_body(rowd_ref, cold_ref, keep_ref, keepc_ref):
    lane = lax.broadcasted_iota(jnp.int32, (B, B), 1)
    sub = lax.broadcasted_iota(jnp.int32, (B, B), 0)
    ident = (lane == sub).astype(jnp.float32)
    lane1 = lax.broadcasted_iota(jnp.int32, (1, B), 1)

    def block_step(i, _):
        row_i = _coords_row(rowd_ref, i)
        alive0 = jnp.ones((1, B), jnp.float32)

        def cross(j, alive):
            col_j = _coords_col(cold_ref, j)
            sup = _pair_suppress(col_j, row_i)
            kj = keepc_ref[pl.ds(pl.multiple_of(j * B, B), B), :][:, 0:1]
            hit = jnp.max(sup * kj, axis=0, keepdims=True)
            return alive * (1.0 - hit)

        alive = lax.fori_loop(0, i, cross, alive0)

        col_i = _coords_col(cold_ref, i)
        s_ii = _pair_suppress(col_i, row_i) * (sub < lane).astype(jnp.float32)

        def greedy(k, alive):
            onehot = (lane1 == k).astype(jnp.float32)
            ak = jnp.sum(alive * onehot)
            rowk = jnp.max(s_ii * (sub == k).astype(jnp.float32), axis=0,
                           keepdims=True)
            return alive * (1.0 - rowk * ak)

        alive = lax.fori_loop(0, B, greedy, alive)

        keep_ref[0:1, pl.ds(pl.multiple_of(i * B, B), B)] = alive
        alive_col = lax.dot_general(ident, alive, (((1,), (1,)), ((), ())),
                                    preferred_element_type=jnp.float32)
        keepc_ref[pl.ds(pl.multiple_of(i * B, B), B), :] = jnp.broadcast_to(
            alive_col, (B, 8))
        return 0

    keep_ref[...] = jnp.ones_like(keep_ref)
    lax.fori_loop(0, NB, block_step, 0)


@jax.jit
def kernel(boxes, labels, scores):
    order = jnp.argsort(-scores)
    sb = boxes[order]
    slab = labels[order].astype(jnp.float32)

    rowd = jnp.zeros((8, NPAD), jnp.float32)
    rowd = rowd.at[0:4, :N].set(sb.T)
    rowd = rowd.at[4, :N].set(slab)
    rowd = rowd.at[4, N:].set(-1.0)
    cold = rowd.T

    keep_row = pl.pallas_call(
        _nms_body,
        out_shape=jax.ShapeDtypeStruct((1, NPAD), jnp.float32),
        scratch_shapes=[pltpu.VMEM((NPAD, 8), jnp.float32)],
    )(rowd, cold)

    keep_sorted = keep_row[0, :N]
    m = jnp.zeros((N,), jnp.float32).at[order].set(keep_sorted)
    return jnp.concatenate([boxes * m[:, None], (scores * m)[:, None]], axis=1)


# speedup vs baseline: 54.7238x; 4.5360x over previous
import jax
import jax.numpy as jnp
from jax import lax
from jax.experimental import pallas as pl
from jax.experimental.pallas import tpu as pltpu
from jax.experimental.pallas import tpu_sc as plsc

N = 5000
NUM_CLASSES = 8
IOU_THRESHOLD = 0.5
L = 16
NIN = 5008
KCAP = 5024

_MESH = plsc.VectorSubcoreMesh(core_axis_name="c", subcore_axis_name="s",
                               num_cores=2, num_subcores=16)


def _lane_max(v, iota):
    for sh in (1, 2, 4, 8):
        v = jnp.maximum(v, jnp.take(v, iota ^ sh))
    return v


def _sc_nms_body(ycs_h, xcs_h, hs_h, ws_h, meta_h, out_h,
                 yc_v, xc_v, h_v, w_v, meta_v,
                 ky1, kx1, ky2, kx2, kar, keep_v):
    cid = lax.axis_index("c")
    sid = lax.axis_index("s")
    wid = sid * 2 + cid

    @pl.when(wid < NUM_CLASSES)
    def _():
        pltpu.sync_copy(ycs_h, yc_v)
        pltpu.sync_copy(xcs_h, xc_v)
        pltpu.sync_copy(hs_h, h_v)
        pltpu.sync_copy(ws_h, w_v)
        pltpu.sync_copy(meta_h, meta_v)

        iota = lax.iota(jnp.int32, L)
        widv = jnp.full((L,), wid, jnp.int32)
        start = plsc.load_gather(meta_v, [widv])[0].astype(jnp.int32)
        cnt = plsc.load_gather(meta_v, [widv + NUM_CLASSES])[0].astype(
            jnp.int32)

        def zero_body(i, carry):
            keep_v[pl.ds(i * L, L)] = jnp.zeros((L,), jnp.float32)
            return carry

        lax.fori_loop(0, NIN // L, zero_body, 0)

        lane0 = iota == 0

        def cand(i, kcnt):
            g = start + i
            gidx = jnp.full((L,), g, jnp.int32)
            ycc = plsc.load_gather(yc_v, [gidx])
            xcc = plsc.load_gather(xc_v, [gidx])
            hc = jnp.abs(plsc.load_gather(h_v, [gidx]))
            wc = jnp.abs(plsc.load_gather(w_v, [gidx]))
            y1c = ycc - 0.5 * hc
            x1c = xcc - 0.5 * wc
            y2c = ycc + 0.5 * hc
            x2c = xcc + 0.5 * wc
            ac = hc * wc

            nch = (kcnt + (L - 1)) // L

            def chunk(cix, hit):
                sl = pl.ds(cix * L, L)
                iy1 = jnp.maximum(ky1[sl], y1c)
                ix1 = jnp.maximum(kx1[sl], x1c)
                iy2 = jnp.minimum(ky2[sl], y2c)
                ix2 = jnp.minimum(kx2[sl], x2c)
                ih = jnp.maximum(iy2 - iy1, 0.0)
                iw = jnp.maximum(ix2 - ix1, 0.0)
                inter = ih * iw
                union = kar[sl] + ac - inter
                iou = inter / jnp.maximum(union, 1e-9)
                valid = (iota + cix * L) < kcnt
                ov = jnp.where(valid & (iou > IOU_THRESHOLD), 1.0, 0.0)
                return jnp.maximum(hit, ov)

            hit = lax.fori_loop(0, nch, chunk, jnp.zeros((L,), jnp.float32))
            sup = _lane_max(hit, iota)[0] > 0.5
            keepf = jnp.where(sup, 0.0, 1.0)

            plsc.store_scatter(keep_v, [gidx],
                               jnp.full((L,), keepf, jnp.float32), mask=lane0)

            amask = lane0 & jnp.logical_not(sup)
            kvec = jnp.full((L,), kcnt, jnp.int32)
            plsc.store_scatter(ky1, [kvec], y1c, mask=amask)
            plsc.store_scatter(kx1, [kvec], x1c, mask=amask)
            plsc.store_scatter(ky2, [kvec], y2c, mask=amask)
            plsc.store_scatter(kx2, [kvec], x2c, mask=amask)
            plsc.store_scatter(kar, [kvec], ac, mask=amask)
            return kcnt + jnp.where(sup, 0, 1)

        lax.fori_loop(0, cnt, cand, jnp.int32(0))

        pltpu.sync_copy(keep_v, out_h.at[wid])


_sc_nms = pl.kernel(
    _sc_nms_body,
    out_type=jax.ShapeDtypeStruct((NUM_CLASSES, NIN), jnp.float32),
    mesh=_MESH,
    compiler_params=pltpu.CompilerParams(needs_layout_passes=False),
    scratch_types=[
        pltpu.VMEM((NIN,), jnp.float32),
        pltpu.VMEM((NIN,), jnp.float32),
        pltpu.VMEM((NIN,), jnp.float32),
        pltpu.VMEM((NIN,), jnp.float32),
        pltpu.VMEM((L,), jnp.float32),
        pltpu.VMEM((KCAP,), jnp.float32),
        pltpu.VMEM((KCAP,), jnp.float32),
        pltpu.VMEM((KCAP,), jnp.float32),
        pltpu.VMEM((KCAP,), jnp.float32),
        pltpu.VMEM((KCAP,), jnp.float32),
        pltpu.VMEM((NIN,), jnp.float32),
    ],
)


@jax.jit
def kernel(boxes, labels, scores):
    lab = labels.astype(jnp.int32)
    order = jnp.lexsort((-scores, lab))
    sb = boxes[order]

    counts = jnp.zeros((NUM_CLASSES,), jnp.int32).at[lab].add(1)
    starts = jnp.concatenate([jnp.zeros((1,), jnp.int32),
                              jnp.cumsum(counts)[:-1].astype(jnp.int32)])
    meta = jnp.concatenate([starts, counts]).astype(jnp.float32)

    def padded(col):
        return jnp.zeros((NIN,), jnp.float32).at[:N].set(col)

    out8 = _sc_nms(padded(sb[:, 0]), padded(sb[:, 1]),
                   padded(sb[:, 2]), padded(sb[:, 3]), meta)

    keep_sorted = jnp.sum(out8, axis=0)[:N]
    m = jnp.zeros((N,), jnp.float32).at[order].set(keep_sorted)
    return jnp.concatenate([boxes * m[:, None], (scores * m)[:, None]], axis=1)


# speedup vs baseline: 58.0577x; 1.0609x over previous
import jax
import jax.numpy as jnp
from jax import lax
from jax.experimental import pallas as pl
from jax.experimental.pallas import tpu as pltpu
from jax.experimental.pallas import tpu_sc as plsc

N = 5000
NUM_CLASSES = 8
IOU_THRESHOLD = 0.5
L = 16
NIN = 5008
KCAP = 5024

_MESH = plsc.VectorSubcoreMesh(core_axis_name="c", subcore_axis_name="s",
                               num_cores=2, num_subcores=16)


def _lane_max(v, iota):
    for sh in (1, 2, 4, 8):
        v = jnp.maximum(v, jnp.take(v, iota ^ sh))
    return v


def _sc_nms_body(ycs_h, xcs_h, hs_h, ws_h, meta_h, out_h,
                 yc_v, xc_v, h_v, w_v, meta_v,
                 ky1, kx1, ky2, kx2, kar, keep_v):
    cid = lax.axis_index("c")
    sid = lax.axis_index("s")

    @pl.when((cid == 0) & (sid < NUM_CLASSES))
    def _():
        wid = sid
        pltpu.sync_copy(ycs_h, yc_v)
        pltpu.sync_copy(xcs_h, xc_v)
        pltpu.sync_copy(hs_h, h_v)
        pltpu.sync_copy(ws_h, w_v)
        pltpu.sync_copy(meta_h, meta_v)

        iota = lax.iota(jnp.int32, L)
        widv = jnp.full((L,), wid, jnp.int32)
        start = plsc.load_gather(meta_v, [widv])[0].astype(jnp.int32)
        cnt = plsc.load_gather(meta_v, [widv + NUM_CLASSES])[0].astype(
            jnp.int32)

        def zero_body(i, carry):
            keep_v[pl.ds(i * L, L)] = jnp.zeros((L,), jnp.float32)
            return carry

        lax.fori_loop(0, NIN // L, zero_body, 0)

        sent1 = jnp.full((L,), 3.0e30, jnp.float32)
        sent2 = jnp.full((L,), -3.0e30, jnp.float32)
        zero = jnp.zeros((L,), jnp.float32)

        def sent_body(i, carry):
            sl = pl.ds(i * L, L)
            ky1[sl] = sent1
            kx1[sl] = sent1
            ky2[sl] = sent2
            kx2[sl] = sent2
            kar[sl] = zero
            return carry

        lax.fori_loop(0, KCAP // L, sent_body, 0)

        lane0 = iota == 0

        def cand(i, kcnt):
            g = start + i
            gidx = jnp.full((L,), g, jnp.int32)
            ycc = plsc.load_gather(yc_v, [gidx])
            xcc = plsc.load_gather(xc_v, [gidx])
            hc = jnp.abs(plsc.load_gather(h_v, [gidx]))
            wc = jnp.abs(plsc.load_gather(w_v, [gidx]))
            y1c = ycc - 0.5 * hc
            x1c = xcc - 0.5 * wc
            y2c = ycc + 0.5 * hc
            x2c = xcc + 0.5 * wc
            ac = hc * wc

            def sweep(sl, hit):
                iy1 = jnp.maximum(ky1[sl], y1c)
                ix1 = jnp.maximum(kx1[sl], x1c)
                iy2 = jnp.minimum(ky2[sl], y2c)
                ix2 = jnp.minimum(kx2[sl], x2c)
                ih = jnp.maximum(iy2 - iy1, 0.0)
                iw = jnp.maximum(ix2 - ix1, 0.0)
                inter = ih * iw
                union = kar[sl] + ac - inter
                iou = inter / jnp.maximum(union, 1e-9)
                return jnp.maximum(hit,
                                   jnp.where(iou > IOU_THRESHOLD, 1.0, 0.0))

            nch2 = (kcnt + (2 * L - 1)) // (2 * L)

            def chunk(cix, hit):
                base = cix * (2 * L)
                hit = sweep(pl.ds(base, L), hit)
                return sweep(pl.ds(base + L, L), hit)

            hit = lax.fori_loop(0, nch2, chunk, jnp.zeros((L,), jnp.float32))
            sup = _lane_max(hit, iota)[0] > 0.5
            keepf = jnp.where(sup, 0.0, 1.0)

            plsc.store_scatter(keep_v, [gidx],
                               jnp.full((L,), keepf, jnp.float32), mask=lane0)

            amask = lane0 & jnp.logical_not(sup)
            kvec = jnp.full((L,), kcnt, jnp.int32)
            plsc.store_scatter(ky1, [kvec], y1c, mask=amask)
            plsc.store_scatter(kx1, [kvec], x1c, mask=amask)
            plsc.store_scatter(ky2, [kvec], y2c, mask=amask)
            plsc.store_scatter(kx2, [kvec], x2c, mask=amask)
            plsc.store_scatter(kar, [kvec], ac, mask=amask)
            return kcnt + jnp.where(sup, 0, 1)

        lax.fori_loop(0, cnt, cand, jnp.int32(0))

        pltpu.sync_copy(keep_v, out_h.at[wid])


_sc_nms = pl.kernel(
    _sc_nms_body,
    out_type=jax.ShapeDtypeStruct((NUM_CLASSES, NIN), jnp.float32),
    mesh=_MESH,
    compiler_params=pltpu.CompilerParams(needs_layout_passes=False),
    scratch_types=[
        pltpu.VMEM((NIN,), jnp.float32),
        pltpu.VMEM((NIN,), jnp.float32),
        pltpu.VMEM((NIN,), jnp.float32),
        pltpu.VMEM((NIN,), jnp.float32),
        pltpu.VMEM((L,), jnp.float32),
        pltpu.VMEM((KCAP,), jnp.float32),
        pltpu.VMEM((KCAP,), jnp.float32),
        pltpu.VMEM((KCAP,), jnp.float32),
        pltpu.VMEM((KCAP,), jnp.float32),
        pltpu.VMEM((KCAP,), jnp.float32),
        pltpu.VMEM((NIN,), jnp.float32),
    ],
)


@jax.jit
def kernel(boxes, labels, scores):
    lab = labels.astype(jnp.int32)
    order = jnp.lexsort((-scores, lab))
    sb = boxes[order]

    counts = jnp.zeros((NUM_CLASSES,), jnp.int32).at[lab].add(1)
    starts = jnp.concatenate([jnp.zeros((1,), jnp.int32),
                              jnp.cumsum(counts)[:-1].astype(jnp.int32)])
    meta = jnp.concatenate([starts, counts]).astype(jnp.float32)

    def padded(col):
        return jnp.zeros((NIN,), jnp.float32).at[:N].set(col)

    out8 = _sc_nms(padded(sb[:, 0]), padded(sb[:, 1]),
                   padded(sb[:, 2]), padded(sb[:, 3]), meta)

    keep_sorted = jnp.sum(out8, axis=0)[:N]
    m = jnp.zeros((N,), jnp.float32).at[order].set(keep_sorted)
    return jnp.concatenate([boxes * m[:, None], (scores * m)[:, None]], axis=1)


# speedup vs baseline: 62.5923x; 1.0781x over previous
import jax
import jax.numpy as jnp
from jax import lax
from jax.experimental import pallas as pl
from jax.experimental.pallas import tpu as pltpu
from jax.experimental.pallas import tpu_sc as plsc

N = 5000
NUM_CLASSES = 8
IOU_THRESHOLD = 0.5
L = 16
NIN = 5008
KCAP = 5024

_MESH = plsc.VectorSubcoreMesh(core_axis_name="c", subcore_axis_name="s",
                               num_cores=2, num_subcores=16)


def _lane_max(v, iota):
    for sh in (1, 2, 4, 8):
        v = jnp.maximum(v, jnp.take(v, iota ^ sh))
    return v


def _sc_nms_body(ycs_h, xcs_h, hs_h, ws_h, order_h, meta_h, out_h,
                 yc_v, xc_v, h_v, w_v, ord_v, meta_v,
                 ky1, kx1, ky2, kx2, kar, keep_v):
    cid = lax.axis_index("c")
    sid = lax.axis_index("s")

    @pl.when((cid == 0) & (sid < NUM_CLASSES))
    def _():
        wid = sid
        pltpu.sync_copy(ycs_h, yc_v)
        pltpu.sync_copy(xcs_h, xc_v)
        pltpu.sync_copy(hs_h, h_v)
        pltpu.sync_copy(ws_h, w_v)
        pltpu.sync_copy(order_h, ord_v)
        pltpu.sync_copy(meta_h, meta_v)

        iota = lax.iota(jnp.int32, L)
        widv = jnp.full((L,), wid, jnp.int32)
        start = plsc.load_gather(meta_v, [widv])[0].astype(jnp.int32)
        cnt = plsc.load_gather(meta_v, [widv + NUM_CLASSES])[0].astype(
            jnp.int32)

        def zero_body(i, carry):
            keep_v[pl.ds(i * L, L)] = jnp.zeros((L,), jnp.float32)
            return carry

        lax.fori_loop(0, NIN // L, zero_body, 0)

        sent1 = jnp.full((L,), 3.0e30, jnp.float32)
        sent2 = jnp.full((L,), -3.0e30, jnp.float32)
        zero = jnp.zeros((L,), jnp.float32)

        def sent_body(i, carry):
            sl = pl.ds(i * L, L)
            ky1[sl] = sent1
            kx1[sl] = sent1
            ky2[sl] = sent2
            kx2[sl] = sent2
            kar[sl] = zero
            return carry

        lax.fori_loop(0, KCAP // L, sent_body, 0)

        lane0 = iota == 0

        def cand(i, kcnt):
            g = start + i
            gidx = jnp.full((L,), g, jnp.int32)
            oidx = plsc.load_gather(ord_v, [gidx]).astype(jnp.int32)
            ycc = plsc.load_gather(yc_v, [oidx])
            xcc = plsc.load_gather(xc_v, [oidx])
            hc = jnp.abs(plsc.load_gather(h_v, [oidx]))
            wc = jnp.abs(plsc.load_gather(w_v, [oidx]))
            y1c = ycc - 0.5 * hc
            x1c = xcc - 0.5 * wc
            y2c = ycc + 0.5 * hc
            x2c = xcc + 0.5 * wc
            ac = hc * wc

            def sweep(sl, hit):
                iy1 = jnp.maximum(ky1[sl], y1c)
                ix1 = jnp.maximum(kx1[sl], x1c)
                iy2 = jnp.minimum(ky2[sl], y2c)
                ix2 = jnp.minimum(kx2[sl], x2c)
                ih = jnp.maximum(iy2 - iy1, 0.0)
                iw = jnp.maximum(ix2 - ix1, 0.0)
                inter = ih * iw
                union = kar[sl] + ac - inter
                iou = inter / jnp.maximum(union, 1e-9)
                return jnp.maximum(hit,
                                   jnp.where(iou > IOU_THRESHOLD, 1.0, 0.0))

            nch2 = (kcnt + (2 * L - 1)) // (2 * L)

            def chunk(cix, hit):
                base = cix * (2 * L)
                hit = sweep(pl.ds(base, L), hit)
                return sweep(pl.ds(base + L, L), hit)

            hit = lax.fori_loop(0, nch2, chunk, jnp.zeros((L,), jnp.float32))
            sup = _lane_max(hit, iota)[0] > 0.5
            keepf = jnp.where(sup, 0.0, 1.0)

            plsc.store_scatter(keep_v, [gidx],
                               jnp.full((L,), keepf, jnp.float32), mask=lane0)

            amask = lane0 & jnp.logical_not(sup)
            kvec = jnp.full((L,), kcnt, jnp.int32)
            plsc.store_scatter(ky1, [kvec], y1c, mask=amask)
            plsc.store_scatter(kx1, [kvec], x1c, mask=amask)
            plsc.store_scatter(ky2, [kvec], y2c, mask=amask)
            plsc.store_scatter(kx2, [kvec], x2c, mask=amask)
            plsc.store_scatter(kar, [kvec], ac, mask=amask)
            return kcnt + jnp.where(sup, 0, 1)

        lax.fori_loop(0, cnt, cand, jnp.int32(0))

        pltpu.sync_copy(keep_v, out_h.at[wid])


_sc_nms = pl.kernel(
    _sc_nms_body,
    out_type=jax.ShapeDtypeStruct((NUM_CLASSES, NIN), jnp.float32),
    mesh=_MESH,
    compiler_params=pltpu.CompilerParams(needs_layout_passes=False),
    scratch_types=[
        pltpu.VMEM((NIN,), jnp.float32),
        pltpu.VMEM((NIN,), jnp.float32),
        pltpu.VMEM((NIN,), jnp.float32),
        pltpu.VMEM((NIN,), jnp.float32),
        pltpu.VMEM((NIN,), jnp.float32),
        pltpu.VMEM((L,), jnp.float32),
        pltpu.VMEM((KCAP,), jnp.float32),
        pltpu.VMEM((KCAP,), jnp.float32),
        pltpu.VMEM((KCAP,), jnp.float32),
        pltpu.VMEM((KCAP,), jnp.float32),
        pltpu.VMEM((KCAP,), jnp.float32),
        pltpu.VMEM((NIN,), jnp.float32),
    ],
)


@jax.jit
def kernel(boxes, labels, scores):
    lab = labels.astype(jnp.int32)
    order = jnp.lexsort((-scores, lab))
    counts = jnp.zeros((NUM_CLASSES,), jnp.int32).at[lab].add(1)
    starts = jnp.concatenate([jnp.zeros((1,), jnp.int32),
                              jnp.cumsum(counts)[:-1].astype(jnp.int32)])
    meta = jnp.concatenate([starts, counts]).astype(jnp.float32)

    ordf = jnp.zeros((NIN,), jnp.float32).at[:N].set(order.astype(jnp.float32))

    def padded(col):
        return jnp.zeros((NIN,), jnp.float32).at[:N].set(col)

    bx = boxes.astype(jnp.float32)
    out8 = _sc_nms(padded(bx[:, 0]), padded(bx[:, 1]), padded(bx[:, 2]),
                   padded(bx[:, 3]), ordf, meta)

    keep_sorted = jnp.sum(out8, axis=0)[:N]
    m = jnp.zeros((N,), jnp.float32).at[order].set(keep_sorted)
    return jnp.concatenate([boxes * m[:, None], (scores * m)[:, None]], axis=1)


# speedup vs baseline: 70.1075x; 1.1201x over previous
import jax
import jax.numpy as jnp
from jax import lax
from jax.experimental import pallas as pl
from jax.experimental.pallas import tpu as pltpu
from jax.experimental.pallas import tpu_sc as plsc

N = 5000
NUM_CLASSES = 8
IOU_THRESHOLD = 0.5
L = 16
NIN = 5008
KCAP = 5024

_MESH = plsc.VectorSubcoreMesh(core_axis_name="c", subcore_axis_name="s",
                               num_cores=2, num_subcores=16)


def _lane_max(v, iota):
    for sh in (1, 2, 4, 8):
        v = jnp.maximum(v, jnp.take(v, iota ^ sh))
    return v


def _sc_nms_body(ycs_h, xcs_h, hs_h, ws_h, order_h, meta_h, out_h,
                 yc_v, xc_v, h_v, w_v, ord_v, meta_v,
                 ky1, kx1, ky2, kx2, kar, keep_v):
    cid = lax.axis_index("c")
    sid = lax.axis_index("s")

    @pl.when((cid == 0) & (sid < NUM_CLASSES))
    def _():
        wid = sid
        pltpu.sync_copy(ycs_h, yc_v)
        pltpu.sync_copy(xcs_h, xc_v)
        pltpu.sync_copy(hs_h, h_v)
        pltpu.sync_copy(ws_h, w_v)
        pltpu.sync_copy(order_h, ord_v)
        pltpu.sync_copy(meta_h, meta_v)

        iota = lax.iota(jnp.int32, L)
        widv = jnp.full((L,), wid, jnp.int32)
        start = plsc.load_gather(meta_v, [widv])[0].astype(jnp.int32)
        cnt = plsc.load_gather(meta_v, [widv + NUM_CLASSES])[0].astype(
            jnp.int32)

        def zero_body(i, carry):
            keep_v[pl.ds(i * L, L)] = jnp.zeros((L,), jnp.float32)
            return carry

        lax.fori_loop(0, NIN // L, zero_body, 0)

        sent1 = jnp.full((L,), 3.0e30, jnp.float32)
        sent2 = jnp.full((L,), -3.0e30, jnp.float32)
        zero = jnp.zeros((L,), jnp.float32)

        def sent_body(i, carry):
            sl = pl.ds(i * L, L)
            ky1[sl] = sent1
            kx1[sl] = sent1
            ky2[sl] = sent2
            kx2[sl] = sent2
            kar[sl] = zero
            return carry

        lax.fori_loop(0, KCAP // L, sent_body, 0)

        lane0 = iota == 0

        def fetch(gidx):
            oidx = plsc.load_gather(ord_v, [gidx]).astype(jnp.int32)
            ycc = plsc.load_gather(yc_v, [oidx])
            xcc = plsc.load_gather(xc_v, [oidx])
            hc = jnp.abs(plsc.load_gather(h_v, [oidx]))
            wc = jnp.abs(plsc.load_gather(w_v, [oidx]))
            return (ycc - 0.5 * hc, xcc - 0.5 * wc,
                    ycc + 0.5 * hc, xcc + 0.5 * wc, hc * wc)

        def iou_gt(y1a, x1a, y2a, x2a, aa, y1b, x1b, y2b, x2b, ab):
            iy1 = jnp.maximum(y1a, y1b)
            ix1 = jnp.maximum(x1a, x1b)
            iy2 = jnp.minimum(y2a, y2b)
            ix2 = jnp.minimum(x2a, x2b)
            ih = jnp.maximum(iy2 - iy1, 0.0)
            iw = jnp.maximum(ix2 - ix1, 0.0)
            inter = ih * iw
            union = aa + ab - inter
            iou = inter / jnp.maximum(union, 1e-9)
            return jnp.where(iou > IOU_THRESHOLD, 1.0, 0.0)

        npairs = (cnt + 1) // 2

        def cand2(p, kcnt):
            g0 = start + 2 * p
            gidx0 = jnp.full((L,), g0, jnp.int32)
            gidx1 = gidx0 + 1
            a = fetch(gidx0)
            b = fetch(gidx1)

            def sweep(sl, hits):
                hA, hB = hits
                k1 = ky1[sl]
                k2 = kx1[sl]
                k3 = ky2[sl]
                k4 = kx2[sl]
                k5 = kar[sl]
                hA = jnp.maximum(hA, iou_gt(k1, k2, k3, k4, k5, *a))
                hB = jnp.maximum(hB, iou_gt(k1, k2, k3, k4, k5, *b))
                return hA, hB

            nch2 = (kcnt + (2 * L - 1)) // (2 * L)

            def chunk(cix, hits):
                base = cix * (2 * L)
                hits = sweep(pl.ds(base, L), hits)
                return sweep(pl.ds(base + L, L), hits)

            z = jnp.zeros((L,), jnp.float32)
            hA, hB = lax.fori_loop(0, nch2, chunk, (z, z))

            supA = _lane_max(hA, iota)[0] > 0.5
            keptA = jnp.logical_not(supA)
            abhit = iou_gt(*a, *b)[0] > 0.5
            validB = (2 * p + 1) < cnt
            supB = (_lane_max(hB, iota)[0] > 0.5) | (keptA & abhit)
            keptB = jnp.logical_not(supB) & validB

            plsc.store_scatter(
                keep_v, [gidx0],
                jnp.full((L,), jnp.where(supA, 0.0, 1.0), jnp.float32),
                mask=lane0)
            plsc.store_scatter(
                keep_v, [gidx1],
                jnp.full((L,), jnp.where(supB, 0.0, 1.0), jnp.float32),
                mask=lane0 & validB)

            ia = jnp.where(keptA, 1, 0)
            amaskA = lane0 & keptA
            amaskB = lane0 & keptB
            kvecA = jnp.full((L,), kcnt, jnp.int32)
            kvecB = kvecA + ia
            plsc.store_scatter(ky1, [kvecA], a[0], mask=amaskA)
            plsc.store_scatter(kx1, [kvecA], a[1], mask=amaskA)
            plsc.store_scatter(ky2, [kvecA], a[2], mask=amaskA)
            plsc.store_scatter(kx2, [kvecA], a[3], mask=amaskA)
            plsc.store_scatter(kar, [kvecA], a[4], mask=amaskA)
            plsc.store_scatter(ky1, [kvecB], b[0], mask=amaskB)
            plsc.store_scatter(kx1, [kvecB], b[1], mask=amaskB)
            plsc.store_scatter(ky2, [kvecB], b[2], mask=amaskB)
            plsc.store_scatter(kx2, [kvecB], b[3], mask=amaskB)
            plsc.store_scatter(kar, [kvecB], b[4], mask=amaskB)
            return kcnt + ia + jnp.where(keptB, 1, 0)

        lax.fori_loop(0, npairs, cand2, jnp.int32(0))

        pltpu.sync_copy(keep_v, out_h.at[wid])


_sc_nms = pl.kernel(
    _sc_nms_body,
    out_type=jax.ShapeDtypeStruct((NUM_CLASSES, NIN), jnp.float32),
    mesh=_MESH,
    compiler_params=pltpu.CompilerParams(needs_layout_passes=False),
    scratch_types=[
        pltpu.VMEM((NIN,), jnp.float32),
        pltpu.VMEM((NIN,), jnp.float32),
        pltpu.VMEM((NIN,), jnp.float32),
        pltpu.VMEM((NIN,), jnp.float32),
        pltpu.VMEM((NIN,), jnp.float32),
        pltpu.VMEM((L,), jnp.float32),
        pltpu.VMEM((KCAP,), jnp.float32),
        pltpu.VMEM((KCAP,), jnp.float32),
        pltpu.VMEM((KCAP,), jnp.float32),
        pltpu.VMEM((KCAP,), jnp.float32),
        pltpu.VMEM((KCAP,), jnp.float32),
        pltpu.VMEM((NIN,), jnp.float32),
    ],
)


@jax.jit
def kernel(boxes, labels, scores):
    lab = labels.astype(jnp.int32)
    order = jnp.lexsort((-scores, lab))
    counts = jnp.zeros((NUM_CLASSES,), jnp.int32).at[lab].add(1)
    starts = jnp.concatenate([jnp.zeros((1,), jnp.int32),
                              jnp.cumsum(counts)[:-1].astype(jnp.int32)])
    meta = jnp.concatenate([starts, counts]).astype(jnp.float32)

    ordf = jnp.zeros((NIN,), jnp.float32).at[:N].set(order.astype(jnp.float32))

    def padded(col):
        return jnp.zeros((NIN,), jnp.float32).at[:N].set(col)

    bx = boxes.astype(jnp.float32)
    out8 = _sc_nms(padded(bx[:, 0]), padded(bx[:, 1]), padded(bx[:, 2]),
                   padded(bx[:, 3]), ordf, meta)

    keep_sorted = jnp.sum(out8, axis=0)[:N]
    m = jnp.zeros((N,), jnp.float32).at[order].set(keep_sorted)
    return jnp.concatenate([boxes * m[:, None], (scores * m)[:, None]], axis=1)
